# Initial kernel scaffold; baseline (speedup 1.0000x reference)
import functools

import jax
import jax.numpy as jnp
from jax.experimental import pallas as pl
from jax.experimental.pallas import tpu as pltpu

CODEBOOK_SIZE = 8192
CODEBOOK_DIM = 256
QUEUE_SIZE = 65536
BETA = 0.25
ALPHA = 1.0
GAMMA = 1.0

_TM = 512


def _argmin_body(z_ref, zsq_ref, e_ref, esq_ref, tok_ref):
    dot = jax.lax.dot_general(
        z_ref[...], e_ref[...],
        dimension_numbers=(((1,), (1,)), ((), ())),
        preferred_element_type=jnp.float32,
    )
    d = (zsq_ref[...] + esq_ref[...]) - 2.0 * dot
    tok = jnp.argmin(d, axis=1).astype(jnp.int32)
    tok_ref[...] = tok[None, None, :]


def _compute_tokens(z_flat, zsq, emb_weight, esq):
    n = z_flat.shape[0]
    nb = n // _TM
    tok3 = pl.pallas_call(
        _argmin_body,
        grid=(nb,),
        in_specs=[
            pl.BlockSpec((_TM, CODEBOOK_DIM), lambda i: (i, 0)),
            pl.BlockSpec((_TM, 1), lambda i: (i, 0)),
            pl.BlockSpec((CODEBOOK_SIZE, CODEBOOK_DIM), lambda i: (0, 0)),
            pl.BlockSpec((1, CODEBOOK_SIZE), lambda i: (0, 0)),
        ],
        out_specs=pl.BlockSpec((1, 1, _TM), lambda i: (i, 0, 0)),
        out_shape=jax.ShapeDtypeStruct((nb, 1, _TM), jnp.int32),
    )(z_flat, zsq, emb_weight, esq.reshape(1, CODEBOOK_SIZE))
    return tok3.reshape(n)


def _calc_wasserstein_loss(zq_feat, c):
    D = zq_feat.shape[1]
    std = jnp.max(jnp.std(c, axis=0, ddof=1))
    z = zq_feat / (std + 1e-08)
    c = c / (std + 1e-08)
    eye = jnp.eye(D, dtype=z.dtype)
    z_mean = jnp.mean(z, axis=0)
    z_cov = jnp.cov(z.T) + 1e-06 * eye
    c_mean = jnp.mean(c, axis=0)
    c_cov = jnp.cov(c.T) + 1e-06 * eye
    diff = z_mean - c_mean
    part_mean = jnp.sum(diff * diff)
    S1, Q1 = jnp.linalg.eigh(z_cov)
    sqrt_S1 = jnp.diag(jnp.sqrt(jax.nn.relu(S1) + 1e-08))
    temp = Q1 @ sqrt_S1
    z_sqrt_cov = temp @ Q1.T
    temp = z_sqrt_cov @ c_cov
    covariance = temp @ z_sqrt_cov
    S2, Q2 = jnp.linalg.eigh(covariance)
    sqrt_S2 = jnp.sqrt(jax.nn.relu(S2) + 1e-08)
    part_cov = jax.nn.relu(jnp.trace(z_cov + c_cov) - 2.0 * jnp.sum(sqrt_S2))
    return jnp.sqrt(part_mean + part_cov + 1e-10)


def kernel(z, emb_weight, queue):
    B, C, H, W = z.shape
    zt = jnp.transpose(z, (0, 2, 3, 1))
    z_flat = zt.reshape(-1, CODEBOOK_DIM)
    N = z_flat.shape[0]
    zsq = jnp.sum(z_flat ** 2, axis=1, keepdims=True)
    esq = jnp.sum(emb_weight ** 2, axis=1)

    token = _compute_tokens(z_flat, zsq, emb_weight, esq)

    queue_new = queue.at[0:N, :].set(z_flat)
    z_q = jnp.take(emb_weight, token, axis=0).reshape(zt.shape)
    w_loss = _calc_wasserstein_loss(queue_new, emb_weight)
    mse1 = jnp.mean((z_q - zt) ** 2)
    mse2 = jnp.mean((z_q - zt) ** 2)
    loss = BETA * mse1 + ALPHA * mse2 + GAMMA * w_loss
    z_q_st = zt + (z_q - zt)
    quant_error = jnp.mean(jnp.sum((z_q - zt) ** 2, axis=3))
    hist = jnp.bincount(token, minlength=CODEBOOK_SIZE, length=CODEBOOK_SIZE).astype(jnp.float32)
    codebook_usage_counts = jnp.sum((hist > 0).astype(jnp.float32))
    codebook_utilization = codebook_usage_counts / CODEBOOK_SIZE
    avg_probs = hist / jnp.sum(hist)
    codebook_perplexity = jnp.exp(-jnp.sum(avg_probs * jnp.log(avg_probs + 1e-10)))
    zq_out = jnp.transpose(z_q_st, (0, 3, 1, 2))
    return (zq_out, loss, w_loss, quant_error, codebook_utilization, codebook_perplexity)


# speedup vs baseline: 4.7585x; 4.7585x over previous
import functools

import jax
import jax.numpy as jnp
from jax import lax
from jax.experimental import pallas as pl
from jax.experimental.pallas import tpu as pltpu
from jax.experimental.pallas import tpu_sc as plsc

K = 8192
D = 256
Q = 65536
N = 16384
BETA = 0.25
ALPHA = 1.0
GAMMA = 1.0

_TM = 256
_NS_ITERS = 16

_NC = 2
_NSUB = 16
_NW = _NC * _NSUB
_BPW = N // _NW
_CH = 128


def _k1a_body(z_ref, zsq_ref, e_ref, esq_ref, d_ref):
    dot = lax.dot_general(z_ref[...], e_ref[...], (((1,), (1,)), ((), ())),
                          preferred_element_type=jnp.float32)
    d_ref[...] = (zsq_ref[...] + esq_ref[...]) - 2.0 * dot


def _k1b_body(d_ref, tok_ref):
    dmat = d_ref[...]
    lmin = jnp.min(dmat, axis=1, keepdims=True)
    iota = lax.broadcasted_iota(jnp.int32, dmat.shape, 1)
    idx = jnp.min(jnp.where(dmat == lmin, iota, 2**30), axis=1)
    tok_ref[...] = idx.astype(jnp.int32)[None, None, :]


def _compute_tokens(z_flat, zsq, emb_weight, esq):
    nb = N // _TM
    d = pl.pallas_call(
        _k1a_body,
        grid=(nb,),
        in_specs=[
            pl.BlockSpec((_TM, D), lambda i: (i, 0)),
            pl.BlockSpec((_TM, 1), lambda i: (i, 0)),
            pl.BlockSpec((K, D), lambda i: (0, 0)),
            pl.BlockSpec((1, K), lambda i: (0, 0)),
        ],
        out_specs=pl.BlockSpec((_TM, K), lambda i: (i, 0)),
        out_shape=jax.ShapeDtypeStruct((N, K), jnp.float32),
    )(z_flat, zsq, emb_weight, esq.reshape(1, K))
    tok3 = pl.pallas_call(
        _k1b_body,
        grid=(nb,),
        in_specs=[pl.BlockSpec((_TM, K), lambda i: (i, 0))],
        out_specs=pl.BlockSpec((1, 1, _TM), lambda i: (i, 0, 0)),
        out_shape=jax.ShapeDtypeStruct((nb, 1, _TM), jnp.int32),
    )(d)
    return tok3.reshape(N)


_BINS = K // _NW


def _sc_body(emb_hbm, tok_hbm, zq_hbm, idx_v, rows_v, sem):
    c = lax.axis_index("c")
    s = lax.axis_index("s")
    wid = c * _NSUB + s
    base = wid * _BPW
    nch = _BPW // _CH

    for h in range(nch):
        pltpu.sync_copy(tok_hbm.at[pl.ds(base + h * _CH, _CH)], idx_v.at[h])
    for h in range(nch):
        pltpu.async_copy(emb_hbm.at[idx_v.at[h]], rows_v, sem).wait()
        pltpu.sync_copy(rows_v, zq_hbm.at[pl.ds(base + h * _CH, _CH)])


def _sc_gather(emb_weight, token):
    mesh = plsc.VectorSubcoreMesh(core_axis_name="c", subcore_axis_name="s")
    kern = functools.partial(
        pl.kernel,
        mesh=mesh,
        out_type=jax.ShapeDtypeStruct((N, D), jnp.float32),
        scratch_types=[
            pltpu.VMEM((_BPW // _CH, _CH), jnp.int32),
            pltpu.VMEM((_CH, D), jnp.float32),
            pltpu.SemaphoreType.DMA,
        ],
    )(_sc_body)
    return kern(emb_weight, token)


def _k3_body(z_ref, q_ref, e_ref, gq_ref, qs_ref, gc_ref, es_ref,
             gq_s, qs_s, gc_s, es_s):
    i = pl.program_id(0)

    @pl.when(i == 0)
    def _():
        gq_s[...] = jnp.zeros_like(gq_s)
        qs_s[...] = jnp.zeros_like(qs_s)
        gc_s[...] = jnp.zeros_like(gc_s)
        es_s[...] = jnp.zeros_like(es_s)

    @pl.when(i < 16)
    def _():
        x = jnp.where(i < 4, z_ref[...], q_ref[...])
        gq_s[...] += lax.dot_general(x, x, (((0,), (0,)), ((), ())),
                                     preferred_element_type=jnp.float32)
        qs_s[...] += jnp.sum(x, axis=0, keepdims=True)

    @pl.when(i >= 16)
    def _():
        xe = e_ref[...]
        gc_s[...] += lax.dot_general(xe, xe, (((0,), (0,)), ((), ())),
                                     preferred_element_type=jnp.float32)
        es_s[...] += jnp.sum(xe, axis=0, keepdims=True)

    @pl.when(i == 17)
    def _():
        gq_ref[...] = gq_s[...]
        qs_ref[...] = qs_s[...]
        gc_ref[...] = gc_s[...]
        es_ref[...] = es_s[...]


def _compute_grams(z_flat, queue, emb_weight):
    nb = 18
    return pl.pallas_call(
        _k3_body,
        grid=(nb,),
        in_specs=[
            pl.BlockSpec((4096, D), lambda i: (jnp.minimum(i, 3), 0)),
            pl.BlockSpec((4096, D), lambda i: (jnp.clip(i, 4, 15), 0)),
            pl.BlockSpec((4096, D), lambda i: (jnp.clip(i - 16, 0, 1), 0)),
        ],
        out_specs=[
            pl.BlockSpec((D, D), lambda i: (0, 0)),
            pl.BlockSpec((1, D), lambda i: (0, 0)),
            pl.BlockSpec((D, D), lambda i: (0, 0)),
            pl.BlockSpec((1, D), lambda i: (0, 0)),
        ],
        out_shape=[
            jax.ShapeDtypeStruct((D, D), jnp.float32),
            jax.ShapeDtypeStruct((1, D), jnp.float32),
            jax.ShapeDtypeStruct((D, D), jnp.float32),
            jax.ShapeDtypeStruct((1, D), jnp.float32),
        ],
        scratch_shapes=[
            pltpu.VMEM((D, D), jnp.float32),
            pltpu.VMEM((1, D), jnp.float32),
            pltpu.VMEM((D, D), jnp.float32),
            pltpu.VMEM((1, D), jnp.float32),
        ],
    )(z_flat, queue, emb_weight)


def _k5_body(z_ref, zq_ref, tok_ref, out_ref, mse_ref, hist_ref, acc, hacc):
    i = pl.program_id(0)
    zb = z_ref[...]
    diff = zq_ref[...] - zb
    out_ref[...] = zb + diff
    part = jnp.sum(diff * diff)

    @pl.when(i == 0)
    def _():
        acc[...] = jnp.zeros((1, 1), jnp.float32)
        hacc[...] = jnp.zeros((1, K), jnp.float32)
    acc[...] = acc[...] + part

    tok = tok_ref[...]
    BC = 512
    for b in range(K // BC):
        bins = lax.broadcasted_iota(jnp.int32, (1, BC), 1) + b * BC
        eq = (tok == bins).astype(jnp.float32)
        hacc[:, b * BC:(b + 1) * BC] += jnp.sum(eq, axis=0, keepdims=True)

    @pl.when(i == pl.num_programs(0) - 1)
    def _():
        mse_ref[...] = acc[...]
        hist_ref[...] = hacc[...]


def _st_mse_hist(z_flat, z_q, tok2):
    nb = 8
    tm = N // nb
    return pl.pallas_call(
        _k5_body,
        grid=(nb,),
        in_specs=[
            pl.BlockSpec((tm, D), lambda i: (i, 0)),
            pl.BlockSpec((tm, D), lambda i: (i, 0)),
            pl.BlockSpec((tm, 1), lambda i: (i, 0)),
        ],
        out_specs=[
            pl.BlockSpec((tm, D), lambda i: (i, 0)),
            pl.BlockSpec((1, 1), lambda i: (0, 0)),
            pl.BlockSpec((1, K), lambda i: (0, 0)),
        ],
        out_shape=[
            jax.ShapeDtypeStruct((N, D), jnp.float32),
            jax.ShapeDtypeStruct((1, 1), jnp.float32),
            jax.ShapeDtypeStruct((1, K), jnp.float32),
        ],
        scratch_shapes=[pltpu.VMEM((1, 1), jnp.float32),
                        pltpu.VMEM((1, K), jnp.float32)],
    )(z_flat, z_q, tok2)


def _k4_body(gq_ref, qs_ref, gc_ref, es_ref, hp_ref, mse_ref,
             loss_ref, w_ref, qe_ref, util_ref, perp_ref):
    f32 = jnp.float32
    Kf, Qf, Df = f32(K), f32(Q), f32(D)
    eye_r = lax.broadcasted_iota(jnp.int32, (D, D), 0)
    eye_c = lax.broadcasted_iota(jnp.int32, (D, D), 1)
    eye = (eye_r == eye_c).astype(f32)

    gq = gq_ref[...]
    gc = gc_ref[...]
    qs = qs_ref[...]
    es = es_ref[...]

    c_mean_raw = es / Kf
    diag_gc = jnp.sum(gc * eye, axis=0, keepdims=True)
    var_c = (diag_gc - Kf * c_mean_raw * c_mean_raw) / (Kf - 1.0)
    std = jnp.sqrt(jnp.max(var_c))
    s = std + 1e-8
    s2 = s * s

    z_mean = qs / (Qf * s)
    c_mean = es / (Kf * s)
    dm = z_mean - c_mean
    part_mean = jnp.sum(dm * dm)

    z_cov = (gq / s2 - Qf * (z_mean.T * z_mean)) / (Qf - 1.0) + 1e-6 * eye
    c_cov = (gc / s2 - Kf * (c_mean.T * c_mean)) / (Kf - 1.0) + 1e-6 * eye
    tr_zc = jnp.sum((z_cov + c_cov) * eye)

    m = lax.dot_general(z_cov, c_cov, (((1,), (0,)), ((), ())),
                        preferred_element_type=f32)
    fro = jnp.sqrt(jnp.sum(m * m))
    y = m / fro
    zz = eye
    for _ in range(_NS_ITERS):
        t = 1.5 * eye - 0.5 * lax.dot_general(zz, y, (((1,), (0,)), ((), ())),
                                              preferred_element_type=f32)
        y = lax.dot_general(y, t, (((1,), (0,)), ((), ())),
                            preferred_element_type=f32)
        zz = lax.dot_general(t, zz, (((1,), (0,)), ((), ())),
                             preferred_element_type=f32)
    tr_sqrt = jnp.sqrt(fro) * jnp.sum(y * eye)

    part_cov = jnp.maximum(tr_zc - 2.0 * tr_sqrt, 0.0)
    w = jnp.sqrt(part_mean + part_cov + 1e-10)

    mse_sum = jnp.sum(mse_ref[...])
    mse = mse_sum / (f32(N) * Df)
    loss = BETA * mse + ALPHA * mse + GAMMA * w
    qe = mse_sum / f32(N)

    hist = hp_ref[...]
    usage = jnp.sum((hist > 0.0).astype(f32))
    util = usage / Kf
    tot = jnp.sum(hist)
    p = hist / tot
    perp = jnp.exp(-jnp.sum(p * jnp.log(p + 1e-10)))

    loss_ref[...] = jnp.full((1, 1), loss, f32)
    w_ref[...] = jnp.full((1, 1), w, f32)
    qe_ref[...] = jnp.full((1, 1), qe, f32)
    util_ref[...] = jnp.full((1, 1), util, f32)
    perp_ref[...] = jnp.full((1, 1), perp, f32)


def _finalize(gq, qsum, gc, esum, hist_parts, mse_sum):
    scal = jax.ShapeDtypeStruct((1, 1), jnp.float32)
    return pl.pallas_call(
        _k4_body,
        in_specs=[
            pl.BlockSpec((D, D), lambda: (0, 0)),
            pl.BlockSpec((1, D), lambda: (0, 0)),
            pl.BlockSpec((D, D), lambda: (0, 0)),
            pl.BlockSpec((1, D), lambda: (0, 0)),
            pl.BlockSpec((1, K), lambda: (0, 0)),
            pl.BlockSpec((1, 1), lambda: (0, 0)),
        ],
        out_specs=[pl.BlockSpec((1, 1), lambda: (0, 0))] * 5,
        out_shape=[scal] * 5,
    )(gq, qsum, gc, esum, hist_parts, mse_sum)


def kernel(z, emb_weight, queue):
    zt = jnp.transpose(z, (0, 2, 3, 1))
    z_flat = zt.reshape(-1, D)
    zsq = jnp.sum(z_flat ** 2, axis=1, keepdims=True)
    esq = jnp.sum(emb_weight ** 2, axis=1)

    token = _compute_tokens(z_flat, zsq, emb_weight, esq)

    z_q = _sc_gather(emb_weight, token)

    zq_st, mse_sum, hist_parts = _st_mse_hist(z_flat, z_q,
                                              token.reshape(N, 1))
    gq, qsum, gc, esum = _compute_grams(z_flat, queue, emb_weight)
    loss, w_loss, qe, util, perp = _finalize(gq, qsum, gc, esum,
                                             hist_parts, mse_sum)

    zq_out = jnp.transpose(zq_st.reshape(16, 32, 32, D), (0, 3, 1, 2))
    return (zq_out, loss.reshape(()), w_loss.reshape(()), qe.reshape(()),
            util.reshape(()), perp.reshape(()))
